# Initial kernel scaffold; baseline (speedup 1.0000x reference)
import jax
import jax.numpy as jnp
from jax.experimental import pallas as pl


def kernel(slices, x0, delta):
    raise NotImplementedError("write your pallas kernel here")


# speedup vs baseline: 2.8150x; 2.8150x over previous
import functools

import jax
import jax.numpy as jnp
from jax import lax
from jax.experimental import pallas as pl
from jax.experimental.pallas import tpu as pltpu
from jax.experimental.pallas import tpu_sc as plsc

T = 64
B = 32
SPATIAL = (64, 64, 64)
N = SPATIAL[0] * SPATIAL[1] * SPATIAL[2] * 2
NC, NS, L = 2, 16, 16
NW = NC * NS
NPW = N // NW
C = 512
NSUB = NPW // C
ROWS = N // C
NSL = C // L
NSTEP = T + B
SLEN = NSTEP + L
X0R = 0
ACC = 1
DBASE = 8
SBASE = DBASE + T
AROWS = SBASE + B


def _sc_body(ts_hbm, rowb_hbm, sb_hbm, sd_hbm, x0_hbm, delta_hbm,
             outf_hbm, outb_hbm,
             arena, snapb, rowv, idxv, sbv, sdv, semf, semb):
    wid = lax.axis_index("s") * NC + lax.axis_index("c")

    pltpu.sync_copy(rowb_hbm, rowv)
    pltpu.sync_copy(sb_hbm, sbv)
    pltpu.sync_copy(sd_hbm, sdv)

    def subchunk(sc, carry):
        g = wid * NSUB + sc
        off = g * C
        pltpu.sync_copy(delta_hbm.at[:, pl.ds(off, C)],
                        arena.at[pl.ds(DBASE, T)])
        pltpu.sync_copy(x0_hbm.at[pl.ds(off, C)], arena.at[X0R])
        for i in range(NSL):
            arena[ACC, pl.ds(i * L, L)] = jnp.zeros((L,), jnp.float32)
        for i in range(B // L):
            idxv[pl.ds(i * L, L)] = rowv[pl.ds(i * L, L)] + g

        def step(k, cr):
            src = sbv[pl.ds(k, L)][0]
            dst = sdv[pl.ds(k, L)][0]
            for i in range(NSL):
                arena[dst, pl.ds(i * L, L)] = (
                    arena[ACC, pl.ds(i * L, L)]
                    + arena[src, pl.ds(i * L, L)])
            return cr

        lax.fori_loop(0, NSTEP, step, jnp.int32(0))
        cpyf = pltpu.make_async_copy(arena.at[pl.ds(SBASE, B)],
                                     outf_hbm.at[idxv], semf)
        cpyf.start()

        def bsub(j, cr):
            for i in range(NSL):
                snapb[j, pl.ds(i * L, L)] = (
                    arena[SBASE + j, pl.ds(i * L, L)]
                    - arena[ACC, pl.ds(i * L, L)])
            return cr

        lax.fori_loop(0, B, bsub, jnp.int32(0))
        cpyb = pltpu.make_async_copy(snapb, outb_hbm.at[idxv], semb)
        cpyb.start()
        cpyf.wait()
        cpyb.wait()
        return carry

    lax.fori_loop(0, NSUB, subchunk, jnp.int32(0))


@jax.jit
def _sc_call(ts, rowb, x0f, d2):
    jb = jnp.arange(B, dtype=jnp.int32)
    jt = jnp.arange(T, dtype=jnp.int32)
    pos_e = ts + jb
    pos_a = jt + jnp.searchsorted(ts, jt, side="right").astype(jnp.int32)
    sb = jnp.zeros(SLEN, jnp.int32).at[pos_e].set(X0R).at[pos_a].set(DBASE + jt)
    sd = jnp.zeros(SLEN, jnp.int32).at[pos_e].set(SBASE + jb).at[pos_a].set(ACC)

    kern = functools.partial(
        pl.kernel,
        out_type=[
            jax.ShapeDtypeStruct((B * ROWS, C), jnp.float32),
            jax.ShapeDtypeStruct((B * ROWS, C), jnp.float32),
        ],
        scratch_types=[
            pltpu.VMEM((AROWS, C), jnp.float32),
            pltpu.VMEM((B, C), jnp.float32),
            pltpu.VMEM((B,), jnp.int32),
            pltpu.VMEM((B,), jnp.int32),
            pltpu.VMEM((SLEN,), jnp.int32),
            pltpu.VMEM((SLEN,), jnp.int32),
            pltpu.SemaphoreType.DMA,
            pltpu.SemaphoreType.DMA,
        ],
        mesh=plsc.VectorSubcoreMesh(core_axis_name="c", subcore_axis_name="s"),
    )(_sc_body)
    return kern(ts, rowb, sb, sd, x0f, d2)


def kernel(slices, x0, delta):
    t_idx = slices[:, 0].astype(jnp.int32)
    order = jnp.argsort(t_idx)
    ts = t_idx[order]
    rowb = (order * ROWS).astype(jnp.int32)
    x0f = x0.reshape(N)
    d2 = delta.reshape(T, N)
    outf, outb = _sc_call(ts, rowb, x0f, d2)
    f = outf.reshape(B, *SPATIAL, 2)
    b = outb.reshape(B, *SPATIAL, 2)
    return (lax.complex(f[..., 0], f[..., 1]),
            lax.complex(b[..., 0], b[..., 1]))


# speedup vs baseline: 13.0775x; 4.6457x over previous
import functools

import jax
import jax.numpy as jnp
from jax import lax
from jax.experimental import pallas as pl
from jax.experimental.pallas import tpu as pltpu
from jax.experimental.pallas import tpu_sc as plsc

T = 64
B = 32
SPATIAL = (64, 64, 64)
N = SPATIAL[0] * SPATIAL[1] * SPATIAL[2] * 2
NC, NS, L = 2, 16, 16
NW = NC * NS
NPW = N // NW
C = 512
NSUB = NPW // C
RW = 128
KR = C // RW
ROWS = N // RW
NSL = C // L
NSTEP = T + B
SLEN = NSTEP + L
X0R = 0
ACC = 1
DBASE = 8
SBASE = DBASE + T
AROWS = SBASE + B


def _sc_body(ts_hbm, rowb_hbm, sb_hbm, sd_hbm, x0_hbm, delta_hbm,
             outf_hbm, outb_hbm,
             arena, snapb, rowv, idxv, sbv, sdv, semf, semb):
    wid = lax.axis_index("s") * NC + lax.axis_index("c")

    pltpu.sync_copy(rowb_hbm, rowv)
    pltpu.sync_copy(sb_hbm, sbv)
    pltpu.sync_copy(sd_hbm, sdv)

    def subchunk(sc, carry):
        g = wid * NSUB + sc
        off = g * C
        pltpu.sync_copy(delta_hbm.at[:, pl.ds(off, C)],
                        arena.at[pl.ds(DBASE, T)])
        pltpu.sync_copy(x0_hbm.at[pl.ds(off, C)], arena.at[X0R])
        for i in range(NSL):
            arena[ACC, pl.ds(i * L, L)] = jnp.zeros((L,), jnp.float32)
        for k in range(KR):
            for i in range(B // L):
                idxv[k, pl.ds(i * L, L)] = rowv[pl.ds(i * L, L)] + (g * KR + k)

        def step(k, cr):
            src = sbv[pl.ds(k, L)][0]
            dst = sdv[pl.ds(k, L)][0]
            for i in range(NSL):
                arena[dst, pl.ds(i * L, L)] = (
                    arena[ACC, pl.ds(i * L, L)]
                    + arena[src, pl.ds(i * L, L)])
            return cr

        lax.fori_loop(0, NSTEP, step, jnp.int32(0))
        cpyfs = [
            pltpu.make_async_copy(
                arena.at[pl.ds(SBASE, B), pl.ds(k * RW, RW)],
                outf_hbm.at[idxv.at[k]], semf)
            for k in range(KR)
        ]
        for cp in cpyfs:
            cp.start()

        def bsub(j, cr):
            for i in range(NSL):
                snapb[j, pl.ds(i * L, L)] = (
                    arena[SBASE + j, pl.ds(i * L, L)]
                    - arena[ACC, pl.ds(i * L, L)])
            return cr

        lax.fori_loop(0, B, bsub, jnp.int32(0))
        cpybs = [
            pltpu.make_async_copy(
                snapb.at[pl.ds(0, B), pl.ds(k * RW, RW)],
                outb_hbm.at[idxv.at[k]], semb)
            for k in range(KR)
        ]
        for cp in cpybs:
            cp.start()
        for cp in cpyfs:
            cp.wait()
        for cp in cpybs:
            cp.wait()
        return carry

    lax.fori_loop(0, NSUB, subchunk, jnp.int32(0))


@jax.jit
def _sc_call(ts, rowb, x0f, d2):
    jb = jnp.arange(B, dtype=jnp.int32)
    jt = jnp.arange(T, dtype=jnp.int32)
    pos_e = ts + jb
    pos_a = jt + jnp.searchsorted(ts, jt, side="right").astype(jnp.int32)
    sb = jnp.zeros(SLEN, jnp.int32).at[pos_e].set(X0R).at[pos_a].set(DBASE + jt)
    sd = jnp.zeros(SLEN, jnp.int32).at[pos_e].set(SBASE + jb).at[pos_a].set(ACC)

    kern = functools.partial(
        pl.kernel,
        out_type=[
            jax.ShapeDtypeStruct((B * ROWS, RW), jnp.float32),
            jax.ShapeDtypeStruct((B * ROWS, RW), jnp.float32),
        ],
        scratch_types=[
            pltpu.VMEM((AROWS, C), jnp.float32),
            pltpu.VMEM((B, C), jnp.float32),
            pltpu.VMEM((B,), jnp.int32),
            pltpu.VMEM((KR, B), jnp.int32),
            pltpu.VMEM((SLEN,), jnp.int32),
            pltpu.VMEM((SLEN,), jnp.int32),
            pltpu.SemaphoreType.DMA,
            pltpu.SemaphoreType.DMA,
        ],
        mesh=plsc.VectorSubcoreMesh(core_axis_name="c", subcore_axis_name="s"),
    )(_sc_body)
    return kern(ts, rowb, sb, sd, x0f, d2)


def kernel(slices, x0, delta):
    t_idx = slices[:, 0].astype(jnp.int32)
    order = jnp.argsort(t_idx)
    ts = t_idx[order]
    rowb = (order * ROWS).astype(jnp.int32)
    x0f = x0.reshape(N)
    d2 = delta.reshape(T, N)
    outf, outb = _sc_call(ts, rowb, x0f, d2)
    f = outf.reshape(B, *SPATIAL, 2)
    b = outb.reshape(B, *SPATIAL, 2)
    return (lax.complex(f[..., 0], f[..., 1]),
            lax.complex(b[..., 0], b[..., 1]))


# speedup vs baseline: 13.4055x; 1.0251x over previous
import functools

import jax
import jax.numpy as jnp
from jax import lax
from jax.experimental import pallas as pl
from jax.experimental.pallas import tpu as pltpu
from jax.experimental.pallas import tpu_sc as plsc

T = 64
B = 32
SPATIAL = (64, 64, 64)
N = SPATIAL[0] * SPATIAL[1] * SPATIAL[2] * 2
NC, NS, L = 2, 16, 16
NW = NC * NS
NPW = N // NW
C = 512
NSUB = NPW // C
RW = 128
KR = C // RW
ROWS = N // RW
NSL = C // L
NSTEP = T + B
SLEN = NSTEP + L
X0R = 0
ACC = 1
DBASE = 8
SBASE = DBASE + T
AROWS = SBASE + B


def _sc_body(emit_b, ts_hbm, rowb_hbm, sb_hbm, sd_hbm, x0_hbm, delta_hbm,
             out_hbm, arena, rowv, idxv, sbv, sdv, sem):
    wid = lax.axis_index("s") * NC + lax.axis_index("c")

    pltpu.sync_copy(rowb_hbm, rowv)
    pltpu.sync_copy(sb_hbm, sbv)
    pltpu.sync_copy(sd_hbm, sdv)

    def subchunk(sc, carry):
        g = wid * NSUB + sc
        off = g * C
        pltpu.sync_copy(delta_hbm.at[:, pl.ds(off, C)],
                        arena.at[pl.ds(DBASE, T)])
        pltpu.sync_copy(x0_hbm.at[pl.ds(off, C)], arena.at[X0R])
        for i in range(NSL):
            arena[ACC, pl.ds(i * L, L)] = jnp.zeros((L,), jnp.float32)
        for k in range(KR):
            for i in range(B // L):
                idxv[k, pl.ds(i * L, L)] = rowv[pl.ds(i * L, L)] + (g * KR + k)

        def step(k, cr):
            src = sbv[pl.ds(k, L)][0]
            dst = sdv[pl.ds(k, L)][0]
            for i in range(NSL):
                arena[dst, pl.ds(i * L, L)] = (
                    arena[ACC, pl.ds(i * L, L)]
                    + arena[src, pl.ds(i * L, L)])
            return cr

        lax.fori_loop(0, NSTEP, step, jnp.int32(0))

        if emit_b:
            def bsub(j, cr):
                for i in range(NSL):
                    arena[SBASE + j, pl.ds(i * L, L)] = (
                        arena[SBASE + j, pl.ds(i * L, L)]
                        - arena[ACC, pl.ds(i * L, L)])
                return cr

            lax.fori_loop(0, B, bsub, jnp.int32(0))

        cpys = [
            pltpu.make_async_copy(
                arena.at[pl.ds(SBASE, B), pl.ds(k * RW, RW)],
                out_hbm.at[idxv.at[k]], sem)
            for k in range(KR)
        ]
        for cp in cpys:
            cp.start()
        for cp in cpys:
            cp.wait()
        return carry

    lax.fori_loop(0, NSUB, subchunk, jnp.int32(0))


def _make_kernel(emit_b):
    return functools.partial(
        pl.kernel,
        out_type=jax.ShapeDtypeStruct((B * ROWS, RW), jnp.float32),
        scratch_types=[
            pltpu.VMEM((AROWS, C), jnp.float32),
            pltpu.VMEM((B,), jnp.int32),
            pltpu.VMEM((KR, B), jnp.int32),
            pltpu.VMEM((SLEN,), jnp.int32),
            pltpu.VMEM((SLEN,), jnp.int32),
            pltpu.SemaphoreType.DMA,
        ],
        mesh=plsc.VectorSubcoreMesh(core_axis_name="c", subcore_axis_name="s"),
    )(functools.partial(_sc_body, emit_b))


@jax.jit
def _sc_call(ts, rowb, x0f, d2):
    jb = jnp.arange(B, dtype=jnp.int32)
    jt = jnp.arange(T, dtype=jnp.int32)
    pos_e = ts + jb
    pos_a = jt + jnp.searchsorted(ts, jt, side="right").astype(jnp.int32)
    sb = jnp.zeros(SLEN, jnp.int32).at[pos_e].set(X0R).at[pos_a].set(DBASE + jt)
    sd = jnp.zeros(SLEN, jnp.int32).at[pos_e].set(SBASE + jb).at[pos_a].set(ACC)

    outf = _make_kernel(False)(ts, rowb, sb, sd, x0f, d2)
    outb = _make_kernel(True)(ts, rowb, sb, sd, x0f, d2)
    return outf, outb


def kernel(slices, x0, delta):
    t_idx = slices[:, 0].astype(jnp.int32)
    order = jnp.argsort(t_idx)
    ts = t_idx[order]
    rowb = (order * ROWS).astype(jnp.int32)
    x0f = x0.reshape(N)
    d2 = delta.reshape(T, N)
    outf, outb = _sc_call(ts, rowb, x0f, d2)
    f = outf.reshape(B, *SPATIAL, 2)
    b = outb.reshape(B, *SPATIAL, 2)
    return (lax.complex(f[..., 0], f[..., 1]),
            lax.complex(b[..., 0], b[..., 1]))
